# baseline (speedup 1.0000x reference)
import functools

import jax
import jax.numpy as jnp
from jax import lax
from jax.experimental import pallas as pl
from jax.experimental.pallas import tpu as pltpu
from jax.experimental.pallas import tpu_sc as plsc

D = 64
SEQ = 200
BATCH = 4096
B_TOTAL = BATCH * SEQ
VOCAB_ROWS = 1000000
NC = 2
NS = 16
NW = NC * NS
B_PER_W = B_TOTAL // NW
CHUNK = 256
NCHUNKS = B_PER_W // CHUNK
SLAB = 8 * CHUNK

_mesh = plsc.VectorSubcoreMesh(core_axis_name="c", subcore_axis_name="s")


@functools.partial(
    pl.kernel,
    mesh=_mesh,
    out_type=jax.ShapeDtypeStruct((B_TOTAL, D), jnp.float32),
    scratch_types=[
        pltpu.VMEM((2 * SLAB,), jnp.int32),
        pltpu.VMEM((CHUNK, 2 * D), jnp.float32),
        pltpu.VMEM((CHUNK, 2 * D), jnp.float32),
        pltpu.VMEM((CHUNK, D), jnp.float32),
        pltpu.VMEM((SEQ, D), jnp.float32),
        pltpu.SemaphoreType.DMA,
        pltpu.SemaphoreType.DMA,
        pltpu.SemaphoreType.DMA,
    ],
)
def _embed(tab_hbm, idx_hbm, pe_hbm, out_hbm,
           idx_v, rows_a, rows_b, out_v, pe_v, sem_a, sem_b, osem):
    wid = lax.axis_index("s") * NC + lax.axis_index("c")
    base = wid * B_PER_W
    pltpu.sync_copy(pe_hbm, pe_v)

    def fire(ci, buf, sem):
        slot = lax.rem(lax.div(ci, 8), 2) * SLAB

        @pl.when(lax.rem(ci, 8) == 0)
        def _():
            cb = pl.multiple_of(base + ci * CHUNK, CHUNK)
            pltpu.sync_copy(idx_hbm.at[pl.ds(cb, SLAB)],
                            idx_v.at[pl.ds(slot, SLAB)])
        part = slot + lax.rem(ci, 8) * CHUNK
        pltpu.async_copy(tab_hbm.at[idx_v.at[pl.ds(part, CHUNK)]], buf, sem)

    def process(ci, buf, sem, s0, first):
        cbase = pl.multiple_of(base + ci * CHUNK, CHUNK)
        pltpu.make_async_copy(
            tab_hbm.at[idx_v.at[pl.ds(0, CHUNK)]], buf, sem).wait()
        @pl.when(jnp.logical_not(first))
        def _():
            pltpu.make_async_copy(
                out_v, out_hbm.at[pl.ds(cbase, CHUNK)], osem).wait()

        def row_body(r, s):
            for j in range(D // 16):
                sl = pl.ds(j * 16, 16)
                out_v[r, sl] = buf[r, sl] + pe_v[s, sl]
            s = s + 1
            return lax.select(s == SEQ, 0, s)

        s_end = lax.fori_loop(0, CHUNK, row_body, s0)
        pltpu.async_copy(out_v, out_hbm.at[pl.ds(cbase, CHUNK)], osem)
        return s_end

    fire(0, rows_a, sem_a)

    def pair_body(k, s):
        c0 = k * 2

        @pl.when(c0 + 1 < NCHUNKS)
        def _():
            fire(c0 + 1, rows_b, sem_b)
        s = process(c0, rows_a, sem_a, s, k == 0)

        @pl.when(c0 + 2 < NCHUNKS)
        def _():
            fire(c0 + 2, rows_a, sem_a)
        s = process(c0 + 1, rows_b, sem_b, s, False)
        return s

    lax.fori_loop(0, NCHUNKS // 2, pair_body, lax.rem(base, SEQ))
    last = pl.multiple_of(base + (NCHUNKS - 1) * CHUNK, CHUNK)
    pltpu.make_async_copy(out_v, out_hbm.at[pl.ds(last, CHUNK)], osem).wait()


def kernel(inputs, table, pos_encoding):
    tab_pad = jnp.pad(table, ((0, 0), (0, D)))
    idx = inputs.reshape(-1).astype(jnp.int32)
    pe = pos_encoding[:SEQ]
    out = _embed(tab_pad, idx, pe)
    return out.reshape(BATCH, SEQ, D)


# speedup vs baseline: 1.0870x; 1.0870x over previous
import functools

import jax
import jax.numpy as jnp
from jax import lax
from jax.experimental import pallas as pl
from jax.experimental.pallas import tpu as pltpu
from jax.experimental.pallas import tpu_sc as plsc

D = 64
SEQ = 200
BATCH = 4096
B_TOTAL = BATCH * SEQ
NC = 2
NS = 16
NW = NC * NS
B_PER_W = B_TOTAL // NW
CHUNK = 128
NCHUNKS = B_PER_W // CHUNK

_mesh = plsc.VectorSubcoreMesh(core_axis_name="c", subcore_axis_name="s")


@functools.partial(
    pl.kernel,
    mesh=_mesh,
    out_type=jax.ShapeDtypeStruct((B_TOTAL, D), jnp.float32),
    scratch_types=[
        pltpu.VMEM((16, 128), jnp.int32),
        pltpu.VMEM((CHUNK, 2 * D), jnp.float32),
        pltpu.VMEM((CHUNK, 2 * D), jnp.float32),
        pltpu.VMEM((CHUNK, D), jnp.float32),
        pltpu.VMEM((CHUNK, D), jnp.float32),
        pltpu.VMEM((SEQ, D), jnp.float32),
        pltpu.SemaphoreType.DMA,
        pltpu.SemaphoreType.DMA,
        pltpu.SemaphoreType.DMA,
        pltpu.SemaphoreType.DMA,
    ],
)
def _embed(tab_hbm, idx_hbm, pe_hbm, out_hbm,
           idx_v, rows_a, rows_b, out_a, out_b, pe_v,
           sem_a, sem_b, osem_a, osem_b):
    wid = lax.axis_index("s") * NC + lax.axis_index("c")
    base = wid * B_PER_W
    pltpu.sync_copy(pe_hbm, pe_v)

    def fire(ci, buf, sem):
        slot = lax.rem(lax.div(ci, 8), 2) * 8

        @pl.when(lax.rem(ci, 8) == 0)
        def _():
            row0 = pl.multiple_of((base + ci * CHUNK) // 128, 8)
            pltpu.sync_copy(idx_hbm.at[pl.ds(row0, 8)],
                            idx_v.at[pl.ds(slot, 8)])
        pltpu.async_copy(tab_hbm.at[idx_v.at[slot + lax.rem(ci, 8)]],
                         buf, sem)

    def process(ci, buf, sem, out_v, osem, s0, first):
        cbase = pl.multiple_of(base + ci * CHUNK, CHUNK)
        pltpu.make_async_copy(tab_hbm.at[idx_v.at[0]], buf, sem).wait()
        @pl.when(jnp.logical_not(first))
        def _():
            pltpu.make_async_copy(
                out_v, out_hbm.at[pl.ds(cbase, CHUNK)], osem).wait()

        def row_body(r, s):
            for j in range(D // 16):
                sl = pl.ds(j * 16, 16)
                out_v[r, sl] = buf[r, sl] + pe_v[s, sl]
            s = s + 1
            return lax.select(s == SEQ, 0, s)

        s_end = lax.fori_loop(0, CHUNK, row_body, s0)
        pltpu.async_copy(out_v, out_hbm.at[pl.ds(cbase, CHUNK)], osem)
        return s_end

    fire(0, rows_a, sem_a)

    def pair_body(k, s):
        c0 = k * 2
        first = k == 0

        @pl.when(c0 + 1 < NCHUNKS)
        def _():
            fire(c0 + 1, rows_b, sem_b)
        s = process(c0, rows_a, sem_a, out_a, osem_a, s, first)

        @pl.when(c0 + 2 < NCHUNKS)
        def _():
            fire(c0 + 2, rows_a, sem_a)
        s = process(c0 + 1, rows_b, sem_b, out_b, osem_b, s, first)
        return s

    lax.fori_loop(0, NCHUNKS // 2, pair_body, lax.rem(base, SEQ))
    last = pl.multiple_of(base + (NCHUNKS - 2) * CHUNK, CHUNK)
    pltpu.make_async_copy(out_a, out_hbm.at[pl.ds(last, CHUNK)], osem_a).wait()
    pltpu.make_async_copy(out_b, out_hbm.at[pl.ds(last, CHUNK)], osem_b).wait()


def kernel(inputs, table, pos_encoding):
    tab_pad = jnp.pad(table, ((0, 0), (0, D)))
    idx = inputs.reshape(B_TOTAL // 128, 128).astype(jnp.int32)
    pe = pos_encoding[:SEQ]
    out = _embed(tab_pad, idx, pe)
    return out.reshape(BATCH, SEQ, D)
